# baseline (speedup 1.0000x reference)
import functools

import jax
import jax.numpy as jnp
from jax import lax
from jax.experimental import pallas as pl
from jax.experimental.pallas import tpu as pltpu
from jax.experimental.pallas import tpu_sc as plsc

_B = 16
_L = 16
_NW = 32


def kernel(x, edge_index, edge_attribute, batch):
    N, D = x.shape
    E, DE = edge_attribute.shape
    idt = edge_index.dtype

    assert DE == _L and D % _L == 0 and _B == _L
    assert E % _NW == 0 and N % _L == 0

    ea_rows = E // _NW
    ei_cols = E // _NW
    x_rows = N // _NW
    x_rem = N - x_rows * _NW
    z_rows = N // _NW
    z_rem = N - z_rows * _NW
    tg = (N // _L) // _NW
    t_cols = tg * _L
    t_rem = N - t_cols * _NW
    tg_rem = t_rem // _L
    b_w = (N // _NW) // 8 * 8
    b_rem = N - b_w * _NW

    mesh = plsc.VectorSubcoreMesh(core_axis_name="c", subcore_axis_name="s")

    @functools.partial(
        pl.kernel,
        out_type=(
            jax.ShapeDtypeStruct((N + _B, D), x.dtype),
            jax.ShapeDtypeStruct((2, E + N), idt),
            jax.ShapeDtypeStruct((E + N, DE), edge_attribute.dtype),
            jax.ShapeDtypeStruct((N + _B,), batch.dtype),
        ),
        mesh=mesh,
        compiler_params=pltpu.CompilerParams(use_tc_tiling_on_sc=False),
        scratch_types=(
            pltpu.VMEM((z_rows, DE), jnp.float32),
            pltpu.VMEM((_B, D), jnp.float32),
            pltpu.VMEM((t_cols,), idt),
            pltpu.VMEM((t_cols,), idt),
            pltpu.VMEM((t_cols,), idt),
            pltpu.VMEM((_L,), idt),
        ),
    )
    def vng_pool(x_hbm, ei_hbm, ea_hbm, b_hbm,
                 xo_hbm, eio_hbm, eao_hbm, bo_hbm,
                 zrow_v, zx_v, t0_v, t1_v, bc_v, ub_v):
        w = lax.axis_index("s") * 2 + lax.axis_index("c")

        ea0 = w * ea_rows
        pltpu.sync_copy(ea_hbm.at[pl.ds(ea0, ea_rows), :],
                        eao_hbm.at[pl.ds(ea0, ea_rows), :])

        c0 = w * ei_cols
        pltpu.sync_copy(ei_hbm.at[0, pl.ds(c0, ei_cols)],
                        eio_hbm.at[0, pl.ds(c0, ei_cols)])
        pltpu.sync_copy(ei_hbm.at[1, pl.ds(c0, ei_cols)],
                        eio_hbm.at[1, pl.ds(c0, ei_cols)])

        xr0 = w * x_rows
        pltpu.sync_copy(x_hbm.at[pl.ds(xr0, x_rows), :],
                        xo_hbm.at[pl.ds(xr0, x_rows), :])

        bb0 = w * b_w
        pltpu.sync_copy(b_hbm.at[pl.ds(bb0, b_w)], bo_hbm.at[pl.ds(bb0, b_w)])

        def zfill(r, carry):
            zrow_v[r, :] = jnp.zeros((_L,), jnp.float32)
            return carry
        lax.fori_loop(0, z_rows, zfill, 0)
        pltpu.sync_copy(zrow_v, eao_hbm.at[pl.ds(E + w * z_rows, z_rows), :])

        tc0 = w * t_cols
        pltpu.sync_copy(b_hbm.at[pl.ds(tc0, t_cols)], bc_v.at[pl.ds(0, t_cols)])

        def tfill(g, carry):
            t0_v[pl.ds(g * _L, _L)] = lax.iota(idt, _L) + (tc0 + g * _L)
            t1_v[pl.ds(g * _L, _L)] = bc_v[pl.ds(g * _L, _L)] + N
            return carry
        lax.fori_loop(0, tg, tfill, 0)
        pltpu.sync_copy(t0_v, eio_hbm.at[0, pl.ds(E + tc0, t_cols)])
        pltpu.sync_copy(t1_v, eio_hbm.at[1, pl.ds(E + tc0, t_cols)])

        @pl.when(w == 0)
        def _():
            pltpu.sync_copy(x_hbm.at[pl.ds(_NW * x_rows, x_rem), :],
                            xo_hbm.at[pl.ds(_NW * x_rows, x_rem), :])

        @pl.when(w == 1)
        def _():
            pltpu.sync_copy(zrow_v.at[pl.ds(0, z_rem), :],
                            eao_hbm.at[pl.ds(E + _NW * z_rows, z_rem), :])

        @pl.when(w == 2)
        def _():
            rc0 = _NW * t_cols
            pltpu.sync_copy(b_hbm.at[pl.ds(rc0, t_rem)], bc_v.at[pl.ds(0, t_rem)])

            def tfill2(g, carry):
                t0_v[pl.ds(g * _L, _L)] = lax.iota(idt, _L) + (rc0 + g * _L)
                t1_v[pl.ds(g * _L, _L)] = bc_v[pl.ds(g * _L, _L)] + N
                return carry
            lax.fori_loop(0, tg_rem, tfill2, 0)
            pltpu.sync_copy(t0_v.at[pl.ds(0, t_rem)],
                            eio_hbm.at[0, pl.ds(E + rc0, t_rem)])
            pltpu.sync_copy(t1_v.at[pl.ds(0, t_rem)],
                            eio_hbm.at[1, pl.ds(E + rc0, t_rem)])

        @pl.when(w == 3)
        def _():
            pltpu.sync_copy(b_hbm.at[pl.ds(_NW * b_w, b_rem)],
                            bo_hbm.at[pl.ds(_NW * b_w, b_rem)])
            ub_v[pl.ds(0, _L)] = lax.iota(idt, _L)
            pltpu.sync_copy(ub_v, bo_hbm.at[pl.ds(N, _B)])

        @pl.when(w == 4)
        def _():
            def zxfill(r, carry):
                for k in range(D // _L):
                    zx_v[r, pl.ds(k * _L, _L)] = jnp.zeros((_L,), jnp.float32)
                return carry
            lax.fori_loop(0, _B, zxfill, 0)
            pltpu.sync_copy(zx_v, xo_hbm.at[pl.ds(N, _B), :])

    return vng_pool(x, edge_index, edge_attribute, batch)


# speedup vs baseline: 3.6515x; 3.6515x over previous
import functools

import jax
import jax.numpy as jnp
from jax import lax
from jax.experimental import pallas as pl
from jax.experimental.pallas import tpu as pltpu
from jax.experimental.pallas import tpu_sc as plsc

_B = 16
_L = 16
_NW = 32


def kernel(x, edge_index, edge_attribute, batch):
    N, D = x.shape
    E, DE = edge_attribute.shape
    idt = edge_index.dtype

    assert DE == _L and D % _L == 0 and _B == _L
    assert E % _NW == 0 and N % _L == 0

    ea_rows = E // _NW
    ei_cols = E // _NW
    x_rows = N // _NW
    x_rem = N - x_rows * _NW
    z_rows = N // _NW
    z_rem = N - z_rows * _NW
    tg = (N // _L) // _NW
    t_cols = tg * _L
    t_rem = N - t_cols * _NW
    tg_rem = t_rem // _L
    b_w = (N // _NW) // 8 * 8
    b_rem = N - b_w * _NW

    mesh = plsc.VectorSubcoreMesh(core_axis_name="c", subcore_axis_name="s")

    ea_chunk = ea_rows // 8

    @functools.partial(
        pl.kernel,
        out_type=(
            jax.ShapeDtypeStruct((N + _B, D), x.dtype),
            jax.ShapeDtypeStruct((2, E + N), idt),
            jax.ShapeDtypeStruct((E + N, DE), edge_attribute.dtype),
            jax.ShapeDtypeStruct((N + _B,), batch.dtype),
        ),
        mesh=mesh,
        compiler_params=pltpu.CompilerParams(use_tc_tiling_on_sc=False),
        scratch_types=(
            pltpu.VMEM((ea_chunk, DE), jnp.float32),
            pltpu.VMEM((ea_chunk, DE), jnp.float32),
            pltpu.VMEM((x_rows, D), jnp.float32),
            pltpu.VMEM((ei_cols,), idt),
            pltpu.VMEM((z_rows, DE), jnp.float32),
            pltpu.VMEM((_B, D), jnp.float32),
            pltpu.VMEM((t_cols,), idt),
            pltpu.VMEM((t_cols,), idt),
            pltpu.VMEM((t_cols,), idt),
            pltpu.VMEM((b_w,), idt),
            pltpu.VMEM((_L,), idt),
            pltpu.SemaphoreType.DMA,
            pltpu.SemaphoreType.DMA,
            pltpu.SemaphoreType.DMA,
            pltpu.SemaphoreType.DMA,
        ),
    )
    def vng_pool(x_hbm, ei_hbm, ea_hbm, b_hbm,
                 xo_hbm, eio_hbm, eao_hbm, bo_hbm,
                 ea0_v, ea1_v, x_v, ei_v, zrow_v, zx_v,
                 t0_v, t1_v, bc_v, bs_v, ub_v,
                 in0_s, in1_s, out0_s, out1_s):
        w = lax.axis_index("s") * 2 + lax.axis_index("c")

        ea0 = w * ea_rows
        bufs = (ea0_v, ea1_v)
        in_sems = (in0_s, in1_s)
        out_sems = (out0_s, out1_s)
        n_chunks = ea_rows // ea_chunk

        def ea_in(k):
            r0 = ea0 + k * ea_chunk
            return pltpu.make_async_copy(ea_hbm.at[pl.ds(r0, ea_chunk), :],
                                         bufs[k % 2], in_sems[k % 2])

        def ea_out(k):
            r0 = ea0 + k * ea_chunk
            return pltpu.make_async_copy(bufs[k % 2],
                                         eao_hbm.at[pl.ds(r0, ea_chunk), :],
                                         out_sems[k % 2])

        ea_in(0).start()
        ea_in(1).start()
        for k in range(n_chunks):
            ea_in(k).wait()
            ea_out(k).start()
            if k + 2 < n_chunks:
                ea_out(k).wait()
                ea_in(k + 2).start()
        ea_out(n_chunks - 2).wait()
        ea_out(n_chunks - 1).wait()

        xr0 = w * x_rows
        pltpu.sync_copy(x_hbm.at[pl.ds(xr0, x_rows), :], x_v)
        pltpu.sync_copy(x_v, xo_hbm.at[pl.ds(xr0, x_rows), :])

        c0 = w * ei_cols
        pltpu.sync_copy(ei_hbm.at[0, pl.ds(c0, ei_cols)], ei_v)
        pltpu.sync_copy(ei_v, eio_hbm.at[0, pl.ds(c0, ei_cols)])
        pltpu.sync_copy(ei_hbm.at[1, pl.ds(c0, ei_cols)], ei_v)
        pltpu.sync_copy(ei_v, eio_hbm.at[1, pl.ds(c0, ei_cols)])

        bb0 = w * b_w
        pltpu.sync_copy(b_hbm.at[pl.ds(bb0, b_w)], bs_v)
        pltpu.sync_copy(bs_v, bo_hbm.at[pl.ds(bb0, b_w)])

        def zfill(r, carry):
            zrow_v[r, :] = jnp.zeros((_L,), jnp.float32)
            return carry
        lax.fori_loop(0, z_rows, zfill, 0)
        pltpu.sync_copy(zrow_v, eao_hbm.at[pl.ds(E + w * z_rows, z_rows), :])

        tc0 = w * t_cols
        pltpu.sync_copy(b_hbm.at[pl.ds(tc0, t_cols)], bc_v.at[pl.ds(0, t_cols)])

        def tfill(g, carry):
            t0_v[pl.ds(g * _L, _L)] = lax.iota(idt, _L) + (tc0 + g * _L)
            t1_v[pl.ds(g * _L, _L)] = bc_v[pl.ds(g * _L, _L)] + N
            return carry
        lax.fori_loop(0, tg, tfill, 0)
        pltpu.sync_copy(t0_v, eio_hbm.at[0, pl.ds(E + tc0, t_cols)])
        pltpu.sync_copy(t1_v, eio_hbm.at[1, pl.ds(E + tc0, t_cols)])

        @pl.when(w == 0)
        def _():
            pltpu.sync_copy(x_hbm.at[pl.ds(_NW * x_rows, x_rem), :],
                            xo_hbm.at[pl.ds(_NW * x_rows, x_rem), :])

        @pl.when(w == 1)
        def _():
            pltpu.sync_copy(zrow_v.at[pl.ds(0, z_rem), :],
                            eao_hbm.at[pl.ds(E + _NW * z_rows, z_rem), :])

        @pl.when(w == 2)
        def _():
            rc0 = _NW * t_cols
            pltpu.sync_copy(b_hbm.at[pl.ds(rc0, t_rem)], bc_v.at[pl.ds(0, t_rem)])

            def tfill2(g, carry):
                t0_v[pl.ds(g * _L, _L)] = lax.iota(idt, _L) + (rc0 + g * _L)
                t1_v[pl.ds(g * _L, _L)] = bc_v[pl.ds(g * _L, _L)] + N
                return carry
            lax.fori_loop(0, tg_rem, tfill2, 0)
            pltpu.sync_copy(t0_v.at[pl.ds(0, t_rem)],
                            eio_hbm.at[0, pl.ds(E + rc0, t_rem)])
            pltpu.sync_copy(t1_v.at[pl.ds(0, t_rem)],
                            eio_hbm.at[1, pl.ds(E + rc0, t_rem)])

        @pl.when(w == 3)
        def _():
            pltpu.sync_copy(b_hbm.at[pl.ds(_NW * b_w, b_rem)],
                            bo_hbm.at[pl.ds(_NW * b_w, b_rem)])
            ub_v[pl.ds(0, _L)] = lax.iota(idt, _L)
            pltpu.sync_copy(ub_v, bo_hbm.at[pl.ds(N, _B)])

        @pl.when(w == 4)
        def _():
            def zxfill(r, carry):
                for k in range(D // _L):
                    zx_v[r, pl.ds(k * _L, _L)] = jnp.zeros((_L,), jnp.float32)
                return carry
            lax.fori_loop(0, _B, zxfill, 0)
            pltpu.sync_copy(zx_v, xo_hbm.at[pl.ds(N, _B), :])

    return vng_pool(x, edge_index, edge_attribute, batch)
